# baseline (speedup 1.0000x reference)
import functools

import jax
import jax.numpy as jnp
from jax import lax
from jax.experimental import pallas as pl
from jax.experimental.pallas import tpu as pltpu, tpu_sc as plsc

_CHUNK = 8


def _gather_kernel(B, D, V, b_per_w, NC):
    mesh = plsc.VectorSubcoreMesh(core_axis_name="c", subcore_axis_name="s")
    n_groups = b_per_w // _CHUNK

    @functools.partial(
        pl.kernel,
        mesh=mesh,
        out_type=jax.ShapeDtypeStruct((D, B), jnp.float32),
        compiler_params=pltpu.CompilerParams(
            disable_bounds_checks=True, needs_layout_passes=False
        ),
        scratch_types=[
            pltpu.VMEM((b_per_w,), jnp.int32),
            pltpu.VMEM((2, _CHUNK, D, 128), jnp.float32),
            pltpu.VMEM((D, b_per_w), jnp.float32),
            pltpu.SemaphoreType.DMA,
            pltpu.SemaphoreType.DMA,
        ],
    )
    def k(tab_hbm, idx_hbm, out_hbm, idx_v, buf_v, cols_v, sem0, sem1):
        wid = lax.axis_index("s") * NC + lax.axis_index("c")
        base = wid * b_per_w
        pltpu.sync_copy(idx_hbm.at[pl.ds(base, b_per_w)], idx_v)
        sems = (sem0, sem1)
        lanes = lax.iota(jnp.int32, 16)

        def extract(vec, half):
            out = []
            for j in range(8 * half, 8 * half + 8):
                s = lax.reduce_sum(
                    jnp.where(lanes == j, vec, 0), axes=(0,)
                )
                out.append(s)
            return out

        def fire(scals, slot):
            for k_, v in enumerate(scals):
                off = pl.multiple_of((v - (v & 127)) * 1, 128)
                pltpu.async_copy(
                    tab_hbm.at[:, pl.ds(off, 128)],
                    buf_v.at[slot, k_],
                    sems[slot],
                )

        def drain(slot):
            pltpu.make_async_copy(
                tab_hbm.at[:, pl.ds(0, _CHUNK * 128)],
                buf_v.at[slot],
                sems[slot],
            ).wait()

        def select(g, scals, slot):
            for k_, v in enumerate(scals):
                j = g * _CHUNK + k_
                lane = jnp.full((16,), v & 127, jnp.int32)
                col = jnp.full((16,), j, jnp.int32)
                for h in range(D // 16):
                    cs = lanes + (16 * h)
                    x = plsc.load_gather(buf_v.at[slot, k_], [cs, lane])
                    plsc.store_scatter(cols_v, [cs, col], x)

        vec0 = idx_v[pl.ds(0, 16)]
        fire(extract(vec0, 0), 0)

        def body(gg, carry):
            vec = idx_v[pl.ds(gg * 16, 16)]
            lo = extract(vec, 0)
            hi = extract(vec, 1)

            fire(hi, 1)
            drain(0)
            select(2 * gg, lo, 0)

            @pl.when(gg + 1 < n_groups // 2)
            def _():
                nvec = idx_v[pl.ds((gg + 1) * 16, 16)]
                fire(extract(nvec, 0), 0)

            drain(1)
            select(2 * gg + 1, hi, 1)
            return carry

        lax.fori_loop(0, n_groups // 2, body, 0)
        pltpu.sync_copy(cols_v, out_hbm.at[:, pl.ds(base, b_per_w)])

    return k


def kernel(inputs, embeddings):
    idx = inputs.astype(jnp.int32)
    (B,) = idx.shape
    V, D = embeddings.shape
    info = plsc.get_sparse_core_info()
    NC, NS = info.num_cores, info.num_subcores
    NW = NC * NS
    b_per_w = B // NW
    outT = _gather_kernel(B, D, V, b_per_w, NC)(embeddings.T, idx)
    return outT.T
